# baseline (speedup 1.0000x reference)
import functools

import jax
import jax.numpy as jnp
from jax.experimental import pallas as pl

_NEG = -1e30
_CHUNK = 24


def _insert(carry, v):
    r1, r2, r3 = carry
    nr1 = jnp.maximum(r1, v)
    m = jnp.minimum(r1, v)
    nr2 = jnp.maximum(r2, m)
    m2 = jnp.minimum(r2, m)
    nr3 = jnp.maximum(r3, m2)
    return nr1, nr2, nr3


def _top3_strip(strip):
    M, w = strip.shape
    n_full = M // _CHUNK
    neg = jnp.full((_CHUNK, w), _NEG, jnp.bfloat16)
    carry = (strip[:_CHUNK], neg, neg)
    for k in range(1, n_full):
        carry = _insert(carry, strip[k * _CHUNK : (k + 1) * _CHUNK])
    rem = M - n_full * _CHUNK
    if rem:
        pad = jnp.full((_CHUNK - rem, w), _NEG, jnp.bfloat16)
        last = jnp.concatenate([strip[n_full * _CHUNK :], pad], axis=0)
        carry = _insert(carry, last)
    cand = jnp.concatenate(carry, axis=0)
    m1 = jnp.max(cand, axis=0)
    msk1 = cand >= m1[None, :]
    c1 = jnp.sum(msk1.astype(jnp.float32), axis=0)
    rest1 = jnp.where(msk1, _NEG, cand)
    m2 = jnp.max(rest1, axis=0)
    msk2 = rest1 >= m2[None, :]
    c2 = jnp.sum(msk2.astype(jnp.float32), axis=0)
    rest2 = jnp.where(msk2, _NEG, rest1)
    m3 = jnp.max(rest2, axis=0)
    k1 = jnp.minimum(c1, 3.0)
    k2 = jnp.minimum(c2, 3.0 - k1)
    k3 = 3.0 - k1 - k2
    m1f = m1.astype(jnp.float32)
    m2f = m2.astype(jnp.float32)
    m3f = m3.astype(jnp.float32)
    return k1 * m1f + k2 * m2f + k3 * m3f


def _top3_colsum(inner):
    return _top3_strip(inner)


def _fused_kernel(q_ref, x2_ref, out_ref, *, bb):
    n_classes = x2_ref.shape[0]
    supn = []
    for j in range(n_classes):
        sup = x2_ref[j]
        cnorm = jnp.sqrt(jnp.sum(sup * sup, axis=0, keepdims=True))
        supn.append((sup / jnp.maximum(cnorm, 1e-12)).astype(jnp.bfloat16))
    hw = q_ref.shape[2]
    qf = jnp.concatenate([q_ref[i] for i in range(bb)], axis=1)
    rnorm = jnp.sqrt(jnp.sum(qf * qf, axis=0))
    rinv = 1.0 / jnp.maximum(rnorm, 1e-12)
    q = qf.astype(jnp.bfloat16)
    tlist = []
    for j in range(n_classes):
        inner = jax.lax.dot_general(
            supn[j], q, (((0,), (0,)), ((), ())),
            preferred_element_type=jnp.float32,
        ).astype(jnp.bfloat16)
        tlist.append(_top3_colsum(inner) * rinv)
    T = jnp.stack(tlist, axis=0)
    lane = jax.lax.broadcasted_iota(jnp.int32, T.shape, 1)
    rows = []
    for i in range(bb):
        m = (lane >= i * hw) & (lane < (i + 1) * hw)
        rows.append(jnp.sum(jnp.where(m, T, 0.0), axis=1))
    out_ref[0] = jnp.stack(rows, axis=0)


def kernel(x1, x2):
    B, C, h, w = x1.shape
    W, _, M = x2.shape
    hw = h * w
    q = x1.reshape(B, C, hw)
    BB = 5
    grid = (B // BB,)
    out = pl.pallas_call(
        functools.partial(_fused_kernel, bb=BB),
        grid=grid,
        in_specs=[
            pl.BlockSpec((BB, C, hw), lambda g: (g, 0, 0)),
            pl.BlockSpec((W, C, M), lambda g: (0, 0, 0)),
        ],
        out_specs=pl.BlockSpec((1, BB, W), lambda g: (g, 0, 0)),
        out_shape=jax.ShapeDtypeStruct((B // BB, BB, W), jnp.float32),
    )(q, x2)
    return out.reshape(B, W)


# speedup vs baseline: 1.0297x; 1.0297x over previous
import functools

import jax
import jax.numpy as jnp
from jax.experimental import pallas as pl

_NEG = -1e30
_CHUNK = 48


def _insert(carry, v):
    r1, r2, r3 = carry
    nr1 = jnp.maximum(r1, v)
    m = jnp.minimum(r1, v)
    nr2 = jnp.maximum(r2, m)
    m2 = jnp.minimum(r2, m)
    nr3 = jnp.maximum(r3, m2)
    return nr1, nr2, nr3


def _top3_strip(strip):
    M, w = strip.shape
    n_full = M // _CHUNK
    chunks = [strip[k * _CHUNK : (k + 1) * _CHUNK] for k in range(n_full)]
    rem = M - n_full * _CHUNK
    if rem:
        pad = jnp.full((_CHUNK - rem, w), _NEG, jnp.bfloat16)
        chunks.append(jnp.concatenate([strip[n_full * _CHUNK :], pad], axis=0))
    if len(chunks) % 2:
        chunks.append(jnp.full((_CHUNK, w), _NEG, jnp.bfloat16))
    his, los = [], []
    for a in range(0, len(chunks), 2):
        his.append(jnp.maximum(chunks[a], chunks[a + 1]))
        los.append(jnp.minimum(chunks[a], chunks[a + 1]))
    neg = jnp.full((_CHUNK, w), _NEG, jnp.bfloat16)
    carry = (his[0], neg, neg)
    for v in his[1:]:
        carry = _insert(carry, v)
    lomax = los[0]
    for v in los[1:]:
        lomax = jnp.maximum(lomax, v)
    cand = jnp.concatenate(list(carry) + [lomax], axis=0)
    m1 = jnp.max(cand, axis=0)
    msk1 = cand >= m1[None, :]
    c1 = jnp.sum(msk1.astype(jnp.float32), axis=0)
    rest1 = jnp.where(msk1, _NEG, cand)
    m2 = jnp.max(rest1, axis=0)
    msk2 = rest1 >= m2[None, :]
    c2 = jnp.sum(msk2.astype(jnp.float32), axis=0)
    rest2 = jnp.where(msk2, _NEG, rest1)
    m3 = jnp.max(rest2, axis=0)
    k1 = jnp.minimum(c1, 3.0)
    k2 = jnp.minimum(c2, 3.0 - k1)
    k3 = 3.0 - k1 - k2
    m1f = m1.astype(jnp.float32)
    m2f = m2.astype(jnp.float32)
    m3f = m3.astype(jnp.float32)
    return k1 * m1f + k2 * m2f + k3 * m3f


def _top3_colsum(inner):
    return _top3_strip(inner)


def _fused_kernel(q_ref, x2_ref, out_ref, *, bb):
    n_classes = x2_ref.shape[0]
    supn = []
    for j in range(n_classes):
        sup = x2_ref[j]
        cnorm = jnp.sqrt(jnp.sum(sup * sup, axis=0, keepdims=True))
        supn.append((sup / jnp.maximum(cnorm, 1e-12)).astype(jnp.bfloat16))
    hw = q_ref.shape[2]
    qf = jnp.concatenate([q_ref[i] for i in range(bb)], axis=1)
    rnorm = jnp.sqrt(jnp.sum(qf * qf, axis=0))
    rinv = 1.0 / jnp.maximum(rnorm, 1e-12)
    q = qf.astype(jnp.bfloat16)
    tlist = []
    for j in range(n_classes):
        inner = jax.lax.dot_general(
            supn[j], q, (((0,), (0,)), ((), ())),
            preferred_element_type=jnp.float32,
        ).astype(jnp.bfloat16)
        tlist.append(_top3_colsum(inner) * rinv)
    T = jnp.stack(tlist, axis=0)
    lane = jax.lax.broadcasted_iota(jnp.int32, T.shape, 1)
    rows = []
    for i in range(bb):
        m = (lane >= i * hw) & (lane < (i + 1) * hw)
        rows.append(jnp.sum(jnp.where(m, T, 0.0), axis=1))
    out_ref[0] = jnp.stack(rows, axis=0)


def kernel(x1, x2):
    B, C, h, w = x1.shape
    W, _, M = x2.shape
    hw = h * w
    q = x1.reshape(B, C, hw)
    BB = 5
    grid = (B // BB,)
    out = pl.pallas_call(
        functools.partial(_fused_kernel, bb=BB),
        grid=grid,
        in_specs=[
            pl.BlockSpec((BB, C, hw), lambda g: (g, 0, 0)),
            pl.BlockSpec((W, C, M), lambda g: (0, 0, 0)),
        ],
        out_specs=pl.BlockSpec((1, BB, W), lambda g: (g, 0, 0)),
        out_shape=jax.ShapeDtypeStruct((B // BB, BB, W), jnp.float32),
    )(q, x2)
    return out.reshape(B, W)


# speedup vs baseline: 1.2532x; 1.2170x over previous
import functools

import jax
import jax.numpy as jnp
from jax.experimental import pallas as pl

_NEG = -1e30
_CHUNK = 32


def _insert(carry, v):
    r1, r2, r3 = carry
    nr1 = jnp.maximum(r1, v)
    m = jnp.minimum(r1, v)
    nr2 = jnp.maximum(r2, m)
    m2 = jnp.minimum(r2, m)
    nr3 = jnp.maximum(r3, m2)
    return nr1, nr2, nr3


def _top3_strip(strip):
    M, w = strip.shape
    n_full = M // _CHUNK
    neg = jnp.full((_CHUNK, w), _NEG, jnp.bfloat16)
    carry = (strip[:_CHUNK], neg, neg)
    for k in range(1, n_full):
        carry = _insert(carry, strip[k * _CHUNK : (k + 1) * _CHUNK])
    rem = M - n_full * _CHUNK
    if rem:
        pad = jnp.full((_CHUNK - rem, w), _NEG, jnp.bfloat16)
        last = jnp.concatenate([strip[n_full * _CHUNK :], pad], axis=0)
        carry = _insert(carry, last)
    cand = jnp.concatenate(carry, axis=0)
    m1 = jnp.max(cand, axis=0)
    msk1 = cand >= m1[None, :]
    c1 = jnp.sum(msk1.astype(jnp.float32), axis=0)
    rest1 = jnp.where(msk1, _NEG, cand)
    m2 = jnp.max(rest1, axis=0)
    msk2 = rest1 >= m2[None, :]
    c2 = jnp.sum(msk2.astype(jnp.float32), axis=0)
    rest2 = jnp.where(msk2, _NEG, rest1)
    m3 = jnp.max(rest2, axis=0)
    k1 = jnp.minimum(c1, 3.0)
    k2 = jnp.minimum(c2, 3.0 - k1)
    k3 = 3.0 - k1 - k2
    m1f = m1.astype(jnp.float32)
    m2f = m2.astype(jnp.float32)
    m3f = m3.astype(jnp.float32)
    return k1 * m1f + k2 * m2f + k3 * m3f


def _top3_colsum(inner):
    return _top3_strip(inner)


def _fused_kernel(q_ref, x2_ref, out_ref, *, bb):
    n_classes = x2_ref.shape[0]
    supn = []
    for j in range(n_classes):
        sup = x2_ref[j]
        cnorm = jnp.sqrt(jnp.sum(sup * sup, axis=0, keepdims=True))
        supn.append((sup / jnp.maximum(cnorm, 1e-12)).astype(jnp.bfloat16))
    hw = q_ref.shape[2]
    qf = jnp.concatenate([q_ref[i] for i in range(bb)], axis=1)
    rnorm = jnp.sqrt(jnp.sum(qf * qf, axis=0))
    rinv = 1.0 / jnp.maximum(rnorm, 1e-12)
    q = qf.astype(jnp.bfloat16)
    tlist = []
    for j in range(n_classes):
        inner = jax.lax.dot_general(
            supn[j], q, (((0,), (0,)), ((), ())),
            preferred_element_type=jnp.float32,
        ).astype(jnp.bfloat16)
        tlist.append(_top3_colsum(inner) * rinv)
    T = jnp.stack(tlist, axis=0)
    lane = jax.lax.broadcasted_iota(jnp.int32, T.shape, 1)
    rows = []
    for i in range(bb):
        m = (lane >= i * hw) & (lane < (i + 1) * hw)
        rows.append(jnp.sum(jnp.where(m, T, 0.0), axis=1))
    out_ref[0] = jnp.stack(rows, axis=0)


def kernel(x1, x2):
    B, C, h, w = x1.shape
    W, _, M = x2.shape
    hw = h * w
    q = x1.reshape(B, C, hw)
    BB = 5
    grid = (B // BB,)
    out = pl.pallas_call(
        functools.partial(_fused_kernel, bb=BB),
        grid=grid,
        in_specs=[
            pl.BlockSpec((BB, C, hw), lambda g: (g, 0, 0)),
            pl.BlockSpec((W, C, M), lambda g: (0, 0, 0)),
        ],
        out_specs=pl.BlockSpec((1, BB, W), lambda g: (g, 0, 0)),
        out_shape=jax.ShapeDtypeStruct((B // BB, BB, W), jnp.float32),
    )(q, x2)
    return out.reshape(B, W)
